# Initial kernel scaffold; baseline (speedup 1.0000x reference)
import jax
import jax.numpy as jnp
from jax.experimental import pallas as pl


def kernel(trajectory, embed_x_w, embed_y_w, type_embed_w):
    raise NotImplementedError("write your pallas kernel here")


# speedup vs baseline: 6.1036x; 6.1036x over previous
import functools

import jax
import jax.numpy as jnp
from jax import lax
from jax.experimental import pallas as pl
from jax.experimental.pallas import tpu as pltpu
from jax.experimental.pallas import tpu_sc as plsc

VOCAB = 1024
D = 128
TRAJ_RANGE = 50.0
ROWS_PER_CHUNK = 128


def _prep_kernel(x_ref, y_ref, t_ref, traj_ref, table_ref, idx_ref):
    table_ref[0:VOCAB, :] = x_ref[...] + t_ref[0:1, :]
    table_ref[VOCAB:2 * VOCAB, :] = y_ref[...] + t_ref[1:2, :]
    v = traj_ref[...]
    f = (v + TRAJ_RANGE) * jnp.float32((VOCAB - 1) / (2.0 * TRAJ_RANGE))
    ii = jnp.clip(f.astype(jnp.int32), 0, VOCAB - 1)
    parity = lax.broadcasted_iota(jnp.int32, v.shape, 1) & 1
    idx_ref[...] = ii + parity * VOCAB


def _prep(embed_x_w, embed_y_w, type_embed_w, traj2d):
    return pl.pallas_call(
        _prep_kernel,
        out_shape=(
            jax.ShapeDtypeStruct((2 * VOCAB, D), jnp.float32),
            jax.ShapeDtypeStruct(traj2d.shape, jnp.int32),
        ),
    )(embed_x_w, embed_y_w, type_embed_w, traj2d)


def _sc_gather(idx_flat, table, n_rows, n_workers):
    rows_per_worker = n_rows // n_workers
    chunks_per_worker = rows_per_worker // ROWS_PER_CHUNK
    mesh = plsc.VectorSubcoreMesh(core_axis_name="c", subcore_axis_name="s")

    @functools.partial(
        pl.kernel,
        mesh=mesh,
        out_type=jax.ShapeDtypeStruct((n_rows, D), jnp.float32),
        scratch_types=[
            pltpu.VMEM((ROWS_PER_CHUNK,), jnp.int32),
            pltpu.VMEM((ROWS_PER_CHUNK, D), jnp.float32),
            pltpu.SemaphoreType.DMA,
        ],
    )
    def k(idx_hbm, table_hbm, out_hbm, idx_v, rows_v, sem):
        wid = lax.axis_index("s") * 2 + lax.axis_index("c")
        base = wid * rows_per_worker

        def body(i, _):
            rbase = base + i * ROWS_PER_CHUNK
            pltpu.sync_copy(idx_hbm.at[pl.ds(rbase, ROWS_PER_CHUNK)], idx_v)
            pltpu.async_copy(table_hbm.at[idx_v], rows_v, sem).wait()
            pltpu.sync_copy(rows_v, out_hbm.at[pl.ds(rbase, ROWS_PER_CHUNK)])
            return ()

        lax.fori_loop(0, chunks_per_worker, body, ())

    return k(idx_flat, table)


def kernel(trajectory, embed_x_w, embed_y_w, type_embed_w):
    B, T, _ = trajectory.shape
    n_rows = B * T * 2
    info = plsc.get_sparse_core_info()
    n_workers = info.num_cores * info.num_subcores
    table, idx = _prep(embed_x_w, embed_y_w, type_embed_w,
                       trajectory.reshape(B, 2 * T))
    out = _sc_gather(idx.reshape(n_rows), table, n_rows, n_workers)
    return out.reshape(B, 2 * T, D)


# speedup vs baseline: 7.3345x; 1.2017x over previous
import functools

import jax
import jax.numpy as jnp
from jax import lax
from jax.experimental import pallas as pl
from jax.experimental.pallas import tpu as pltpu
from jax.experimental.pallas import tpu_sc as plsc

VOCAB = 1024
D = 128
TRAJ_RANGE = 50.0
CHUNK = 128
NBUF = 4
AHEAD = 2


def _prep_kernel(x_ref, y_ref, t_ref, traj_ref, table_ref, idx_ref):
    table_ref[0:VOCAB, :] = x_ref[...] + t_ref[0:1, :]
    table_ref[VOCAB:2 * VOCAB, :] = y_ref[...] + t_ref[1:2, :]
    v = traj_ref[...]
    f = (v + TRAJ_RANGE) * jnp.float32((VOCAB - 1) / (2.0 * TRAJ_RANGE))
    ii = jnp.clip(f.astype(jnp.int32), 0, VOCAB - 1)
    parity = lax.broadcasted_iota(jnp.int32, v.shape, 1) & 1
    idx_ref[...] = ii + parity * VOCAB


def _prep(embed_x_w, embed_y_w, type_embed_w, traj2d):
    return pl.pallas_call(
        _prep_kernel,
        out_shape=(
            jax.ShapeDtypeStruct((2 * VOCAB, D), jnp.float32),
            jax.ShapeDtypeStruct(traj2d.shape, jnp.int32),
        ),
    )(embed_x_w, embed_y_w, type_embed_w, traj2d)


def _sc_gather(idx3d, table, n_rows, n_workers):
    rows_per_worker = n_rows // n_workers
    n_chunks = rows_per_worker // CHUNK
    mesh = plsc.VectorSubcoreMesh(core_axis_name="c", subcore_axis_name="s")

    @functools.partial(
        pl.kernel,
        mesh=mesh,
        out_type=jax.ShapeDtypeStruct((n_rows, D), jnp.float32),
        scratch_types=(
            [pltpu.VMEM((n_chunks, CHUNK), jnp.int32)]
            + [pltpu.VMEM((CHUNK, D), jnp.float32) for _ in range(NBUF)]
            + [pltpu.SemaphoreType.DMA for _ in range(2 * NBUF)]
        ),
    )
    def k(idx_hbm, table_hbm, out_hbm, idx_all, *bufs_and_sems):
        rows = bufs_and_sems[:NBUF]
        sg = bufs_and_sems[NBUF:2 * NBUF]
        so = bufs_and_sems[2 * NBUF:3 * NBUF]
        wid = lax.axis_index("s") * 2 + lax.axis_index("c")
        base = wid * rows_per_worker

        pltpu.sync_copy(idx_hbm.at[wid], idx_all)
        for b in range(AHEAD):
            pltpu.async_copy(table_hbm.at[idx_all.at[b]], rows[b], sg[b])

        def visit(g, b):
            ci = g * NBUF + b
            tg = ci + AHEAD
            bg = (b + AHEAD) % NBUF

            @pl.when(jnp.logical_and(tg >= NBUF, tg < n_chunks))
            def _wait_write_free():
                pltpu.make_async_copy(
                    rows[bg], out_hbm.at[pl.ds(0, CHUNK)], so[bg]).wait()

            @pl.when(tg < n_chunks)
            def _start_gather():
                pltpu.async_copy(table_hbm.at[idx_all.at[tg]], rows[bg], sg[bg])

            pltpu.make_async_copy(
                table_hbm.at[idx_all.at[0]], rows[b], sg[b]).wait()
            pltpu.async_copy(
                rows[b], out_hbm.at[pl.ds(base + ci * CHUNK, CHUNK)], so[b])

        def body(g, _):
            for b in range(NBUF):
                visit(g, b)
            return ()

        lax.fori_loop(0, n_chunks // NBUF, body, ())
        for ci in range(n_chunks - NBUF, n_chunks):
            b = ci % NBUF
            pltpu.make_async_copy(
                rows[b], out_hbm.at[pl.ds(0, CHUNK)], so[b]).wait()

    return k(idx3d, table)


def kernel(trajectory, embed_x_w, embed_y_w, type_embed_w):
    B, T, _ = trajectory.shape
    n_rows = B * T * 2
    info = plsc.get_sparse_core_info()
    n_workers = info.num_cores * info.num_subcores
    table, idx = _prep(embed_x_w, embed_y_w, type_embed_w,
                       trajectory.reshape(B, 2 * T))
    rows_per_worker = n_rows // n_workers
    idx3d = idx.reshape(n_workers, rows_per_worker // CHUNK, CHUNK)
    out = _sc_gather(idx3d, table, n_rows, n_workers)
    return out.reshape(B, 2 * T, D)


# speedup vs baseline: 19.7859x; 2.6976x over previous
import functools

import jax
import jax.numpy as jnp
from jax import lax
from jax.experimental import pallas as pl
from jax.experimental.pallas import tpu as pltpu
from jax.experimental.pallas import tpu_sc as plsc

VOCAB = 1024
D = 128
TRAJ_RANGE = 50.0
CHUNK = 128
NBUF = 2
AHEAD = 1


def _prep_kernel(x_ref, y_ref, t_ref, traj_ref, table_ref, idx_ref):
    table_ref[0:VOCAB, :] = x_ref[...] + t_ref[0:1, :]
    table_ref[VOCAB:2 * VOCAB, :] = y_ref[...] + t_ref[1:2, :]
    v = traj_ref[...]
    f = (v + TRAJ_RANGE) * jnp.float32((VOCAB - 1) / (2.0 * TRAJ_RANGE))
    ii = jnp.clip(f.astype(jnp.int32), 0, VOCAB - 1)
    parity = lax.broadcasted_iota(jnp.int32, v.shape, 1) & 1
    idx_ref[...] = ii + parity * VOCAB


def _prep(embed_x_w, embed_y_w, type_embed_w, traj2d):
    return pl.pallas_call(
        _prep_kernel,
        out_shape=(
            jax.ShapeDtypeStruct((2 * VOCAB, D), jnp.float32),
            jax.ShapeDtypeStruct(traj2d.shape, jnp.int32),
        ),
    )(embed_x_w, embed_y_w, type_embed_w, traj2d)


def _sc_gather(idx3d, table, n_rows, n_workers):
    rows_per_worker = n_rows // n_workers
    n_chunks = rows_per_worker // CHUNK
    mesh = plsc.VectorSubcoreMesh(core_axis_name="c", subcore_axis_name="s")

    @functools.partial(
        pl.kernel,
        mesh=mesh,
        out_type=jax.ShapeDtypeStruct((n_rows, D), jnp.float32),
        scratch_types=(
            [pltpu.VMEM((n_chunks, CHUNK), jnp.int32),
             pltpu.VMEM_SHARED((2 * VOCAB, D), jnp.float32)]
            + [pltpu.VMEM((CHUNK, D), jnp.float32) for _ in range(NBUF)]
            + [pltpu.SemaphoreType.DMA for _ in range(2 * NBUF)]
        ),
    )
    def k(idx_hbm, table_hbm, out_hbm, idx_all, spm_table, *bufs_and_sems):
        rows = bufs_and_sems[:NBUF]
        sg = bufs_and_sems[NBUF:2 * NBUF]
        so = bufs_and_sems[2 * NBUF:3 * NBUF]
        sid = lax.axis_index("s")
        wid = sid * 2 + lax.axis_index("c")
        base = wid * rows_per_worker

        tslice = 2 * VOCAB // 16
        pltpu.sync_copy(table_hbm.at[pl.ds(sid * tslice, tslice)],
                        spm_table.at[pl.ds(sid * tslice, tslice)])
        plsc.subcore_barrier()

        pltpu.sync_copy(idx_hbm.at[wid], idx_all)
        for b in range(AHEAD):
            pltpu.async_copy(spm_table.at[idx_all.at[b]], rows[b], sg[b])

        def visit(g, b):
            ci = g * NBUF + b
            tg = ci + AHEAD
            bg = (b + AHEAD) % NBUF

            @pl.when(jnp.logical_and(tg >= NBUF, tg < n_chunks))
            def _wait_write_free():
                pltpu.make_async_copy(
                    rows[bg], out_hbm.at[pl.ds(0, CHUNK)], so[bg]).wait()

            @pl.when(tg < n_chunks)
            def _start_gather():
                pltpu.async_copy(spm_table.at[idx_all.at[tg]], rows[bg], sg[bg])

            pltpu.make_async_copy(
                spm_table.at[idx_all.at[0]], rows[b], sg[b]).wait()
            pltpu.async_copy(
                rows[b], out_hbm.at[pl.ds(base + ci * CHUNK, CHUNK)], so[b])

        def body(g, _):
            for b in range(NBUF):
                visit(g, b)
            return ()

        lax.fori_loop(0, n_chunks // NBUF, body, ())
        for ci in range(n_chunks - NBUF, n_chunks):
            b = ci % NBUF
            pltpu.make_async_copy(
                rows[b], out_hbm.at[pl.ds(0, CHUNK)], so[b]).wait()

    return k(idx3d, table)


def kernel(trajectory, embed_x_w, embed_y_w, type_embed_w):
    B, T, _ = trajectory.shape
    n_rows = B * T * 2
    info = plsc.get_sparse_core_info()
    n_workers = info.num_cores * info.num_subcores
    table, idx = _prep(embed_x_w, embed_y_w, type_embed_w,
                       trajectory.reshape(B, 2 * T))
    rows_per_worker = n_rows // n_workers
    idx3d = idx.reshape(n_workers, rows_per_worker // CHUNK, CHUNK)
    out = _sc_gather(idx3d, table, n_rows, n_workers)
    return out.reshape(B, 2 * T, D)


# speedup vs baseline: 20.2308x; 1.0225x over previous
import functools

import jax
import jax.numpy as jnp
from jax import lax
from jax.experimental import pallas as pl
from jax.experimental.pallas import tpu as pltpu
from jax.experimental.pallas import tpu_sc as plsc

VOCAB = 1024
D = 128
TRAJ_RANGE = 50.0
CHUNK = 128
NBUF = 4
AHEAD = 2
PF = 3


def _prep_kernel(x_ref, y_ref, t_ref, traj_ref, table_ref, idx_ref):
    table_ref[0:VOCAB, :] = x_ref[...] + t_ref[0:1, :]
    table_ref[VOCAB:2 * VOCAB, :] = y_ref[...] + t_ref[1:2, :]
    v = traj_ref[...]
    f = (v + TRAJ_RANGE) * jnp.float32((VOCAB - 1) / (2.0 * TRAJ_RANGE))
    ii = jnp.clip(f.astype(jnp.int32), 0, VOCAB - 1)
    parity = lax.broadcasted_iota(jnp.int32, v.shape, 1) & 1
    idx_ref[...] = ii + parity * VOCAB


def _prep(embed_x_w, embed_y_w, type_embed_w, traj2d):
    return pl.pallas_call(
        _prep_kernel,
        out_shape=(
            jax.ShapeDtypeStruct((2 * VOCAB, D), jnp.float32),
            jax.ShapeDtypeStruct(traj2d.shape, jnp.int32),
        ),
    )(embed_x_w, embed_y_w, type_embed_w, traj2d)


def _sc_gather(idx2d, table, n_rows, n_workers):
    rows_per_worker = n_rows // n_workers
    n_chunks = rows_per_worker // CHUNK
    mesh = plsc.VectorSubcoreMesh(core_axis_name="c", subcore_axis_name="s")

    @functools.partial(
        pl.kernel,
        mesh=mesh,
        out_type=jax.ShapeDtypeStruct((n_rows, D), jnp.float32),
        scratch_types=(
            [pltpu.VMEM((NBUF, CHUNK), jnp.int32),
             pltpu.VMEM_SHARED((2 * VOCAB, D), jnp.float32)]
            + [pltpu.VMEM((CHUNK, D), jnp.float32) for _ in range(NBUF)]
            + [pltpu.SemaphoreType.DMA for _ in range(3 * NBUF)]
        ),
    )
    def k(idx_hbm, table_hbm, out_hbm, idxbuf, spm_table, *bufs_and_sems):
        rows = bufs_and_sems[:NBUF]
        sg = bufs_and_sems[NBUF:2 * NBUF]
        so = bufs_and_sems[2 * NBUF:3 * NBUF]
        si = bufs_and_sems[3 * NBUF:4 * NBUF]
        sid = lax.axis_index("s")
        wid = sid * 2 + lax.axis_index("c")
        base = wid * rows_per_worker
        ibase = wid * n_chunks

        tslice = 2 * VOCAB // 16
        pltpu.sync_copy(table_hbm.at[pl.ds(sid * tslice, tslice)],
                        spm_table.at[pl.ds(sid * tslice, tslice)])
        plsc.subcore_barrier()

        def start_idx(c, slot):
            pltpu.async_copy(idx_hbm.at[ibase + c], idxbuf.at[slot], si[slot])

        def wait_idx(slot):
            pltpu.make_async_copy(
                idx_hbm.at[0], idxbuf.at[slot], si[slot]).wait()

        def start_gather(c, slot):
            pltpu.async_copy(spm_table.at[idxbuf.at[slot]], rows[slot],
                             sg[slot])

        def wait_gather(slot):
            pltpu.make_async_copy(
                spm_table.at[idxbuf.at[slot]], rows[slot], sg[slot]).wait()

        def wait_write(slot):
            pltpu.make_async_copy(
                rows[slot], out_hbm.at[pl.ds(0, CHUNK)], so[slot]).wait()

        for c in range(PF):
            start_idx(c, c % NBUF)
        for c in range(AHEAD):
            wait_idx(c % NBUF)
            start_gather(c, c % NBUF)

        def visit(g, b):
            ci = g * NBUF + b
            tg = ci + AHEAD
            tp = ci + PF
            bg = (b + AHEAD) % NBUF
            bp = (b + PF) % NBUF

            @pl.when(tp < n_chunks)
            def _prefetch_idx():
                start_idx(tp, bp)

            @pl.when(jnp.logical_and(tg >= NBUF, tg < n_chunks))
            def _wait_write_free():
                wait_write(bg)

            @pl.when(jnp.logical_and(tg >= AHEAD, tg < n_chunks))
            def _start_gather():
                wait_idx(bg)
                start_gather(tg, bg)

            wait_gather(b)
            pltpu.async_copy(
                rows[b], out_hbm.at[pl.ds(base + ci * CHUNK, CHUNK)], so[b])

        def body(g, _):
            for b in range(NBUF):
                visit(g, b)
            return ()

        lax.fori_loop(0, n_chunks // NBUF, body, ())
        for ci in range(n_chunks - NBUF, n_chunks):
            wait_write(ci % NBUF)

    return k(idx2d, table)


def kernel(trajectory, embed_x_w, embed_y_w, type_embed_w):
    B, T, _ = trajectory.shape
    n_rows = B * T * 2
    info = plsc.get_sparse_core_info()
    n_workers = info.num_cores * info.num_subcores
    table, idx = _prep(embed_x_w, embed_y_w, type_embed_w,
                       trajectory.reshape(B, 2 * T))
    idx2d = idx.reshape(n_rows // CHUNK, CHUNK)
    out = _sc_gather(idx2d, table, n_rows, n_workers)
    return out.reshape(B, 2 * T, D)
